# Initial kernel scaffold; baseline (speedup 1.0000x reference)
import jax
import jax.numpy as jnp
from jax.experimental import pallas as pl


def kernel(frames):
    raise NotImplementedError("write your pallas kernel here")


# speedup vs baseline: 1.6126x; 1.6126x over previous
import numpy as np
import jax
import jax.numpy as jnp
from jax.experimental import pallas as pl

_ALPHA = 4


def _copy_body(in_ref, out_ref):
    out_ref[...] = in_ref[...]


def kernel(frames):
    C, T, H, W = frames.shape
    n_slow = T // _ALPHA
    idx = np.linspace(0.0, T - 1, n_slow).astype(np.int32)
    assert all(int(v) == (i * (T - 1)) // (n_slow - 1) for i, v in enumerate(idx))

    slow = pl.pallas_call(
        _copy_body,
        grid=(n_slow,),
        in_specs=[
            pl.BlockSpec((C, 1, H, W), lambda i: (0, (i * (T - 1)) // (n_slow - 1), 0, 0)),
        ],
        out_specs=pl.BlockSpec((C, 1, H, W), lambda i: (0, i, 0, 0)),
        out_shape=jax.ShapeDtypeStruct((C, n_slow, H, W), frames.dtype),
    )(frames)

    return (slow, frames)


# speedup vs baseline: 1.6303x; 1.0110x over previous
import numpy as np
import jax
import jax.numpy as jnp
from jax.experimental import pallas as pl

_ALPHA = 4


def _make_body(T, n_slow):
    def body(in_ref, fast_ref, slow_ref):
        i = pl.program_id(0)
        fast_ref[...] = in_ref[...]
        j = ((i + 1) * (n_slow - 1) - 1) // (T - 1)
        sel = (j * (T - 1)) // (n_slow - 1) == i

        @pl.when(sel)
        def _():
            slow_ref[...] = in_ref[...]

    return body


def kernel(frames):
    C, T, H, W = frames.shape
    n_slow = T // _ALPHA
    idx = np.linspace(0.0, T - 1, n_slow).astype(np.int32)
    assert all(int(v) == (j * (T - 1)) // (n_slow - 1) for j, v in enumerate(idx))
    idx_set = set(idx.tolist())
    for i in range(T):
        j = ((i + 1) * (n_slow - 1) - 1) // (T - 1)
        assert ((j * (T - 1)) // (n_slow - 1) == i) == (i in idx_set)

    fast, slow = pl.pallas_call(
        _make_body(T, n_slow),
        grid=(T,),
        in_specs=[
            pl.BlockSpec((C, 1, H, W), lambda i: (0, i, 0, 0)),
        ],
        out_specs=[
            pl.BlockSpec((C, 1, H, W), lambda i: (0, i, 0, 0)),
            pl.BlockSpec(
                (C, 1, H, W),
                lambda i: (0, ((i + 1) * (n_slow - 1) - 1) // (T - 1), 0, 0),
            ),
        ],
        out_shape=[
            jax.ShapeDtypeStruct((C, T, H, W), frames.dtype),
            jax.ShapeDtypeStruct((C, n_slow, H, W), frames.dtype),
        ],
    )(frames)

    return (slow, fast)


# speedup vs baseline: 1.8637x; 1.1431x over previous
import numpy as np
import jax
import jax.numpy as jnp
from jax.experimental import pallas as pl

_ALPHA = 4


def _make_body(T, n_slow):
    def body(in_ref, fast_ref, slow_ref):
        j = pl.program_id(0)
        fast_ref[...] = in_ref[...]
        off = (j * (T - 1)) // (n_slow - 1) - _ALPHA * j
        slow_ref[...] = in_ref[:, pl.ds(off, 1)]

    return body


def kernel(frames):
    C, T, H, W = frames.shape
    n_slow = T // _ALPHA
    idx = np.linspace(0.0, T - 1, n_slow).astype(np.int32)
    assert all(int(v) == (j * (T - 1)) // (n_slow - 1) for j, v in enumerate(idx))
    for j, v in enumerate(idx):
        assert _ALPHA * j <= int(v) < _ALPHA * (j + 1)

    fast, slow = pl.pallas_call(
        _make_body(T, n_slow),
        grid=(n_slow,),
        in_specs=[
            pl.BlockSpec((C, _ALPHA, H, W), lambda j: (0, j, 0, 0)),
        ],
        out_specs=[
            pl.BlockSpec((C, _ALPHA, H, W), lambda j: (0, j, 0, 0)),
            pl.BlockSpec((C, 1, H, W), lambda j: (0, j, 0, 0)),
        ],
        out_shape=[
            jax.ShapeDtypeStruct((C, T, H, W), frames.dtype),
            jax.ShapeDtypeStruct((C, n_slow, H, W), frames.dtype),
        ],
    )(frames)

    return (slow, fast)
